# baseline (speedup 1.0000x reference)
import functools

import jax
import jax.numpy as jnp
from jax import lax
from jax.experimental import pallas as pl
from jax.experimental.pallas import tpu as pltpu
from jax.experimental.pallas import tpu_sc as plsc

DIM = 64
CHUNK = 128
NSLOT = 4
HIST = 50
LANES = 16


@functools.lru_cache(maxsize=None)
def _make_lookup(n_idx: int, n_ent: int, dim: int):
    info = plsc.get_sparse_core_info()
    nw = info.num_cores * info.num_subcores
    batch = n_idx // HIST
    bpw = batch // nw
    assert bpw == NSLOT * CHUNK

    mesh = plsc.VectorSubcoreMesh(core_axis_name="c", subcore_axis_name="s")

    @functools.partial(
        pl.kernel,
        mesh=mesh,
        out_type=jax.ShapeDtypeStruct((HIST, dim, batch), jnp.float32),
        scratch_types=[
            pltpu.VMEM((HIST, bpw), jnp.int32),
            *[pltpu.VMEM((CHUNK, dim), jnp.float32) for _ in range(NSLOT)],
            *[pltpu.VMEM((dim, CHUNK), jnp.float32) for _ in range(NSLOT)],
            pltpu.SemaphoreType.DMA((NSLOT,)),
            pltpu.SemaphoreType.DMA((NSLOT,)),
        ],
        compiler_params=pltpu.CompilerParams(
            use_tc_tiling_on_sc=False, needs_layout_passes=False
        ),
    )
    def lookup(idx_hbm, table_hbm, out_hbm, idx_v, *bufs):
        rows = bufs[:NSLOT]
        outs = bufs[NSLOT : 2 * NSLOT]
        sem_g, sem_w = bufs[2 * NSLOT], bufs[2 * NSLOT + 1]
        wid = lax.axis_index("s") * info.num_cores + lax.axis_index("c")
        b0 = wid * bpw
        pltpu.sync_copy(idx_hbm.at[:, pl.ds(b0, bpw)], idx_v)

        iota = lax.iota(jnp.int32, LANES)

        def transpose(src, dst):
            def g_body(g, carry):
                row_idx = iota + g * LANES
                for c in range(dim):
                    v = plsc.load_gather(
                        src, [row_idx, jnp.full((LANES,), c, jnp.int32)]
                    )
                    dst[c, pl.ds(g * LANES, LANES)] = v
                return carry

            lax.fori_loop(0, CHUNK // LANES, g_body, 0)

        def outer(h, carry):
            for s in range(NSLOT):
                @pl.when(h > 0)
                def _wait_prev_write():
                    pltpu.make_async_copy(
                        outs[s],
                        out_hbm.at[0, pl.ds(0, dim), pl.ds(0, CHUNK)],
                        sem_w.at[s],
                    ).wait()

                pltpu.async_copy(
                    table_hbm.at[idx_v.at[h, pl.ds(s * CHUNK, CHUNK)]],
                    rows[s],
                    sem_g.at[s],
                )
            for s in range(NSLOT):
                pltpu.make_async_copy(
                    table_hbm.at[pl.ds(0, CHUNK)], rows[s], sem_g.at[s]
                ).wait()
                transpose(rows[s], outs[s])
                pltpu.async_copy(
                    outs[s],
                    out_hbm.at[h, pl.ds(0, dim), pl.ds(b0 + s * CHUNK, CHUNK)],
                    sem_w.at[s],
                )
            return carry

        lax.fori_loop(0, HIST, outer, 0)
        for s in range(NSLOT):
            pltpu.make_async_copy(
                outs[s],
                out_hbm.at[0, pl.ds(0, dim), pl.ds(0, CHUNK)],
                sem_w.at[s],
            ).wait()

    return lookup


@jax.jit
def kernel(entities, ent_embs):
    batch, hist = entities.shape
    n_ent, dim = ent_embs.shape
    idx_t = jnp.transpose(entities).astype(jnp.int32)
    out = _make_lookup(batch * hist, n_ent, dim)(idx_t, ent_embs)
    return jnp.transpose(out, (2, 0, 1))


# speedup vs baseline: 2.0803x; 2.0803x over previous
import functools

import jax
import jax.numpy as jnp
from jax import lax
from jax.experimental import pallas as pl
from jax.experimental.pallas import tpu as pltpu
from jax.experimental.pallas import tpu_sc as plsc

DIM = 64
CHUNK = 128
PADW = CHUNK + 1
NSLOT = 4
HIST = 50
LANES = 16


@functools.lru_cache(maxsize=None)
def _make_lookup(n_idx: int, n_ent: int, dim: int):
    info = plsc.get_sparse_core_info()
    nw = info.num_cores * info.num_subcores
    batch = n_idx // HIST
    bpw = batch // nw
    assert bpw == NSLOT * CHUNK

    mesh = plsc.VectorSubcoreMesh(core_axis_name="c", subcore_axis_name="s")

    @functools.partial(
        pl.kernel,
        mesh=mesh,
        out_type=jax.ShapeDtypeStruct((HIST, dim, batch), jnp.float32),
        scratch_types=[
            pltpu.VMEM((HIST, bpw), jnp.int32),
            *[pltpu.VMEM((CHUNK, dim), jnp.float32) for _ in range(NSLOT)],
            *[pltpu.VMEM((dim, PADW), jnp.float32) for _ in range(NSLOT)],
            pltpu.SemaphoreType.DMA((NSLOT,)),
            pltpu.SemaphoreType.DMA((NSLOT,)),
        ],
        compiler_params=pltpu.CompilerParams(
            use_tc_tiling_on_sc=False, needs_layout_passes=False
        ),
    )
    def lookup(idx_hbm, table_hbm, out_hbm, idx_v, *bufs):
        rows = bufs[:NSLOT]
        outs = bufs[NSLOT : 2 * NSLOT]
        sem_g, sem_w = bufs[2 * NSLOT], bufs[2 * NSLOT + 1]
        wid = lax.axis_index("s") * info.num_cores + lax.axis_index("c")
        b0 = wid * bpw
        pltpu.sync_copy(idx_hbm.at[:, pl.ds(b0, bpw)], idx_v)

        iota = lax.iota(jnp.int32, LANES)

        def transpose(src, dst):
            @plsc.parallel_loop(0, CHUNK, unroll=4)
            def _row(r):
                for c0 in range(0, dim, LANES):
                    v = src[r, pl.ds(c0, LANES)]
                    plsc.store_scatter(
                        dst, [iota + c0, jnp.full((LANES,), r, jnp.int32)], v
                    )

        def outer(h, carry):
            for s in range(NSLOT):
                @pl.when(h > 0)
                def _wait_prev_write():
                    pltpu.make_async_copy(
                        outs[s].at[:, pl.ds(0, CHUNK)],
                        out_hbm.at[0, pl.ds(0, dim), pl.ds(0, CHUNK)],
                        sem_w.at[s],
                    ).wait()

                pltpu.async_copy(
                    table_hbm.at[idx_v.at[h, pl.ds(s * CHUNK, CHUNK)]],
                    rows[s],
                    sem_g.at[s],
                )
            for s in range(NSLOT):
                pltpu.make_async_copy(
                    table_hbm.at[pl.ds(0, CHUNK)], rows[s], sem_g.at[s]
                ).wait()
                transpose(rows[s], outs[s])
                pltpu.async_copy(
                    outs[s].at[:, pl.ds(0, CHUNK)],
                    out_hbm.at[h, pl.ds(0, dim), pl.ds(b0 + s * CHUNK, CHUNK)],
                    sem_w.at[s],
                )
            return carry

        lax.fori_loop(0, HIST, outer, 0)
        for s in range(NSLOT):
            pltpu.make_async_copy(
                outs[s].at[:, pl.ds(0, CHUNK)],
                out_hbm.at[0, pl.ds(0, dim), pl.ds(0, CHUNK)],
                sem_w.at[s],
            ).wait()

    return lookup


@jax.jit
def kernel(entities, ent_embs):
    batch, hist = entities.shape
    n_ent, dim = ent_embs.shape
    idx_t = jnp.transpose(entities).astype(jnp.int32)
    out = _make_lookup(batch * hist, n_ent, dim)(idx_t, ent_embs)
    return jnp.transpose(out, (2, 0, 1))
